# baseline (speedup 1.0000x reference)
import jax
import jax.numpy as jnp
from jax import lax
from jax.experimental import pallas as pl

_T = 320
_C = 1024
_K = 48
_KPAD = 128
_BO = 256
_G = _C // _BO


def _shift3(y):
    z = jnp.zeros((y.shape[0], 1), jnp.float32)
    return (jnp.concatenate([z, y[:, :_T - 1]], axis=1),
            y,
            jnp.concatenate([y[:, 1:], z], axis=1))


def _stack_body(x_ref, o_ref):
    r, m, l = _shift3(x_ref[...])
    o_ref[:, 0, :] = r
    o_ref[:, 1, :] = m
    o_ref[:, 2, :] = l


def _shift_stack(x):
    return pl.pallas_call(
        _stack_body,
        grid=(_G,),
        in_specs=[pl.BlockSpec((_BO, _T), lambda i: (i, 0))],
        out_specs=pl.BlockSpec((_BO, 3, _T), lambda i: (i, 0, 0)),
        out_shape=jax.ShapeDtypeStruct((_C, 3, _T), jnp.float32),
    )(x)


def _conv_stack_body(w_ref, x_ref, b_ref, o_ref):
    acc = jnp.dot(w_ref[...], x_ref[...], preferred_element_type=jnp.float32)
    y = jnp.maximum(acc + b_ref[...], 0.0)
    r, m, l = _shift3(y)
    o_ref[:, 0, :] = r
    o_ref[:, 1, :] = m
    o_ref[:, 2, :] = l


def _conv_body(w_ref, x_ref, b_ref, o_ref):
    acc = jnp.dot(w_ref[...], x_ref[...], preferred_element_type=jnp.float32)
    o_ref[...] = jnp.maximum(acc + b_ref[...], 0.0)


def _conv_relu_stack(xcat, wn, b):
    return pl.pallas_call(
        _conv_stack_body,
        grid=(_G,),
        in_specs=[
            pl.BlockSpec((_BO, 3 * _C), lambda i: (i, 0)),
            pl.BlockSpec((3 * _C, _T), lambda i: (0, 0)),
            pl.BlockSpec((_BO, 1), lambda i: (i, 0)),
        ],
        out_specs=pl.BlockSpec((_BO, 3, _T), lambda i: (i, 0, 0)),
        out_shape=jax.ShapeDtypeStruct((_C, 3, _T), jnp.float32),
    )(wn, xcat, b)


def _conv_relu(xcat, wn, b):
    return pl.pallas_call(
        _conv_body,
        grid=(_G,),
        in_specs=[
            pl.BlockSpec((_BO, 3 * _C), lambda i: (i, 0)),
            pl.BlockSpec((3 * _C, _T), lambda i: (0, 0)),
            pl.BlockSpec((_BO, 1), lambda i: (i, 0)),
        ],
        out_specs=pl.BlockSpec((_BO, _T), lambda i: (i, 0)),
        out_shape=jax.ShapeDtypeStruct((_C, _T), jnp.float32),
    )(wn, xcat, b)


def _tail_body(h_ref, ws_ref, bs_ref, o_ref):
    h = h_ref[...]
    ws = ws_ref[...]
    s = jnp.dot(ws.astype(jnp.bfloat16), h.astype(jnp.bfloat16),
                preferred_element_type=jnp.float32) + bs_ref[...]
    csum = jnp.sum(h, axis=0, keepdims=True)

    u_iota = lax.broadcasted_iota(jnp.int32, (_T, _T), 0)
    t_iota = lax.broadcasted_iota(jnp.int32, (_T, _T), 1)
    eye = (u_iota == t_iota).astype(jnp.float32)
    diag = eye * s
    scol = jnp.dot(diag, jnp.ones((_T, _T), jnp.float32),
                   preferred_element_type=jnp.float32,
                   precision=lax.Precision.HIGHEST)
    cmp = (scol >= s).astype(jnp.float32)
    cnt = jnp.sum(cmp, axis=0, keepdims=True)
    low = jnp.max(jnp.where(cnt >= _K, s, -1e30), axis=1, keepdims=True)

    maskf = (s >= low).astype(jnp.float32)
    nz = maskf * (csum != 0.0).astype(jnp.float32)
    ltm = (u_iota <= t_iota).astype(jnp.float32)
    prefix = jnp.dot(nz, ltm, preferred_element_type=jnp.float32,
                     precision=lax.Precision.HIGHEST)
    count = jnp.sum(nz, axis=1, keepdims=True)

    j_iota = lax.broadcasted_iota(jnp.int32, (_KPAD, _T), 0).astype(jnp.float32)
    t2 = lax.broadcasted_iota(jnp.int32, (_KPAD, _T), 1)
    sel = ((prefix == j_iota + 1.0) & (nz > 0.0)).astype(jnp.float32)
    pad = ((j_iota + 1.0 > count) & (t2 == 0)).astype(jnp.float32) * maskf
    pt = sel + pad
    o_ref[...] = lax.dot_general(h, pt, (((1,), (1,)), ((), ())),
                                 preferred_element_type=jnp.float32,
                                 precision=lax.Precision.HIGHEST)


def _tail(h2, ws_row, bs11):
    return pl.pallas_call(
        _tail_body,
        in_specs=[
            pl.BlockSpec((_C, _T), lambda: (0, 0)),
            pl.BlockSpec((1, _C), lambda: (0, 0)),
            pl.BlockSpec((1, 1), lambda: (0, 0)),
        ],
        out_specs=pl.BlockSpec((_C, _KPAD), lambda: (0, 0)),
        out_shape=jax.ShapeDtypeStruct((_C, _KPAD), jnp.float32),
    )(h2, ws_row, bs11)


def kernel(x, W1, b1, W2, b2, Ws, bs):
    X = x[0]
    w1n = W1.reshape(_C, 3 * _C)
    w2n = W2.reshape(_C, 3 * _C)
    xc = _shift_stack(X).reshape(3 * _C, _T)
    h1c = _conv_relu_stack(xc, w1n, b1.reshape(_C, 1)).reshape(3 * _C, _T)
    h2 = _conv_relu(h1c, w2n, b2.reshape(_C, 1))
    out = _tail(h2, Ws.reshape(1, _C), bs.reshape(1, 1))
    return out[None, :, :_K]


# speedup vs baseline: 4.9503x; 4.9503x over previous
import jax
import jax.numpy as jnp
from jax import lax
from jax.experimental import pallas as pl

_T = 320
_C = 1024
_K = 48
_KPAD = 128
_BO = 256
_G = _C // _BO


def _conv_body(w_ref, x_ref, b_ref, o_ref):
    xv = x_ref[...]
    z = jnp.zeros((_C, 1), jnp.float32)
    xr = jnp.concatenate([z, xv[:, :_T - 1]], axis=1)
    xl = jnp.concatenate([xv[:, 1:], z], axis=1)
    acc = jnp.dot(w_ref[0], xr, preferred_element_type=jnp.float32)
    acc += jnp.dot(w_ref[1], xv, preferred_element_type=jnp.float32)
    acc += jnp.dot(w_ref[2], xl, preferred_element_type=jnp.float32)
    o_ref[...] = jnp.maximum(acc + b_ref[...], 0.0)


def _conv_relu(x, wt, b):
    return pl.pallas_call(
        _conv_body,
        grid=(_G,),
        in_specs=[
            pl.BlockSpec((3, _BO, _C), lambda i: (0, i, 0)),
            pl.BlockSpec((_C, _T), lambda i: (0, 0)),
            pl.BlockSpec((_BO, 1), lambda i: (i, 0)),
        ],
        out_specs=pl.BlockSpec((_BO, _T), lambda i: (i, 0)),
        out_shape=jax.ShapeDtypeStruct((_C, _T), jnp.float32),
    )(wt, x, b)


def _tail_body(h_ref, ws_ref, bs_ref, o_ref):
    h = h_ref[...]
    ws = ws_ref[...]
    s = jnp.dot(ws.astype(jnp.bfloat16), h.astype(jnp.bfloat16),
                preferred_element_type=jnp.float32) + bs_ref[...]
    csum = jnp.sum(h, axis=0, keepdims=True)

    u_iota = lax.broadcasted_iota(jnp.int32, (_T, _T), 0)
    t_iota = lax.broadcasted_iota(jnp.int32, (_T, _T), 1)
    eye = (u_iota == t_iota).astype(jnp.float32)
    diag = eye * s
    scol = jnp.dot(diag, jnp.ones((_T, _T), jnp.float32),
                   preferred_element_type=jnp.float32,
                   precision=lax.Precision.HIGHEST)
    cmp = (scol >= s).astype(jnp.float32)
    cnt = jnp.sum(cmp, axis=0, keepdims=True)
    low = jnp.max(jnp.where(cnt >= _K, s, -1e30), axis=1, keepdims=True)

    maskf = (s >= low).astype(jnp.float32)
    nz = maskf * (csum != 0.0).astype(jnp.float32)
    ltm = (u_iota <= t_iota).astype(jnp.float32)
    prefix = jnp.dot(nz, ltm, preferred_element_type=jnp.float32,
                     precision=lax.Precision.HIGHEST)
    count = jnp.sum(nz, axis=1, keepdims=True)

    j_iota = lax.broadcasted_iota(jnp.int32, (_KPAD, _T), 0).astype(jnp.float32)
    t2 = lax.broadcasted_iota(jnp.int32, (_KPAD, _T), 1)
    sel = ((prefix == j_iota + 1.0) & (nz > 0.0)).astype(jnp.float32)
    pad = ((j_iota + 1.0 > count) & (t2 == 0)).astype(jnp.float32) * maskf
    pt = sel + pad
    o_ref[...] = lax.dot_general(h, pt, (((1,), (1,)), ((), ())),
                                 preferred_element_type=jnp.float32,
                                 precision=lax.Precision.HIGHEST)


def _tail(h2, ws_row, bs11):
    return pl.pallas_call(
        _tail_body,
        in_specs=[
            pl.BlockSpec((_C, _T), lambda: (0, 0)),
            pl.BlockSpec((1, _C), lambda: (0, 0)),
            pl.BlockSpec((1, 1), lambda: (0, 0)),
        ],
        out_specs=pl.BlockSpec((_C, _KPAD), lambda: (0, 0)),
        out_shape=jax.ShapeDtypeStruct((_C, _KPAD), jnp.float32),
    )(h2, ws_row, bs11)


def kernel(x, W1, b1, W2, b2, Ws, bs):
    X = x[0]
    w1t = W1.transpose(2, 0, 1)
    w2t = W2.transpose(2, 0, 1)
    h1 = _conv_relu(X, w1t, b1.reshape(_C, 1))
    h2 = _conv_relu(h1, w2t, b2.reshape(_C, 1))
    out = _tail(h2, Ws.reshape(1, _C), bs.reshape(1, 1))
    return out[None, :, :_K]


# speedup vs baseline: 5.3980x; 1.0904x over previous
import jax
import jax.numpy as jnp
from jax import lax
from jax.experimental import pallas as pl
from jax.experimental.pallas import tpu as pltpu

_T = 320
_C = 1024
_K = 48
_KPAD = 128
_BO = 256
_G = _C // _BO


def _conv3(w_ref, xv, b_ref):
    z = jnp.zeros((_C, 1), jnp.float32)
    xr = jnp.concatenate([z, xv[:, :_T - 1]], axis=1)
    xl = jnp.concatenate([xv[:, 1:], z], axis=1)
    acc = jnp.dot(w_ref[0], xr, preferred_element_type=jnp.float32)
    acc += jnp.dot(w_ref[1], xv, preferred_element_type=jnp.float32)
    acc += jnp.dot(w_ref[2], xl, preferred_element_type=jnp.float32)
    return jnp.maximum(acc + b_ref[...], 0.0)


def _tail_compute(h, ws, bs, o_ref):
    s = jnp.dot(ws.astype(jnp.bfloat16), h.astype(jnp.bfloat16),
                preferred_element_type=jnp.float32) + bs
    csum = jnp.sum(h, axis=0, keepdims=True)

    u_iota = lax.broadcasted_iota(jnp.int32, (_T, _T), 0)
    t_iota = lax.broadcasted_iota(jnp.int32, (_T, _T), 1)
    eye = (u_iota == t_iota).astype(jnp.float32)
    diag = eye * s
    scol = jnp.dot(diag, jnp.ones((_T, _T), jnp.float32),
                   preferred_element_type=jnp.float32,
                   precision=lax.Precision.HIGHEST)
    cmp = (scol >= s).astype(jnp.float32)
    cnt = jnp.sum(cmp, axis=0, keepdims=True)
    low = jnp.max(jnp.where(cnt >= _K, s, -1e30), axis=1, keepdims=True)

    maskf = (s >= low).astype(jnp.float32)
    nz = maskf * (csum != 0.0).astype(jnp.float32)
    ltm = (u_iota <= t_iota).astype(jnp.float32)
    prefix = jnp.dot(nz, ltm, preferred_element_type=jnp.float32,
                     precision=lax.Precision.HIGHEST)
    count = jnp.sum(nz, axis=1, keepdims=True)

    j_iota = lax.broadcasted_iota(jnp.int32, (_KPAD, _T), 0).astype(jnp.float32)
    t2 = lax.broadcasted_iota(jnp.int32, (_KPAD, _T), 1)
    sel = ((prefix == j_iota + 1.0) & (nz > 0.0)).astype(jnp.float32)
    pad = ((j_iota + 1.0 > count) & (t2 == 0)).astype(jnp.float32) * maskf
    pt = sel + pad
    g = lax.dot_general(h, pt, (((1,), (1,)), ((), ())),
                        preferred_element_type=jnp.float32,
                        precision=lax.Precision.HIGHEST)
    o_ref[...] = g[:, :_K]


def _fused_body(w1_ref, w2_ref, b1_ref, b2_ref, x_ref, ws_ref, bs_ref,
                o_ref, h1s, h2s):
    p = pl.program_id(0)
    i = pl.program_id(1)

    @pl.when(p == 0)
    def _():
        h1s[pl.ds(i * _BO, _BO), :] = _conv3(w1_ref, x_ref[...], b1_ref)

    @pl.when(p == 1)
    def _():
        h2s[pl.ds(i * _BO, _BO), :] = _conv3(w2_ref, h1s[...], b2_ref)

    @pl.when((p == 1) & (i == _G - 1))
    def _():
        _tail_compute(h2s[...], ws_ref[...], bs_ref[...], o_ref)


def kernel(x, W1, b1, W2, b2, Ws, bs):
    X = x[0]
    w1t = W1.transpose(2, 0, 1)
    w2t = W2.transpose(2, 0, 1)
    out = pl.pallas_call(
        _fused_body,
        grid=(2, _G),
        in_specs=[
            pl.BlockSpec((3, _BO, _C), lambda p, i: (0, i * (1 - p) + (_G - 1) * p, 0)),
            pl.BlockSpec((3, _BO, _C), lambda p, i: (0, i * p, 0)),
            pl.BlockSpec((_BO, 1), lambda p, i: (i * (1 - p) + (_G - 1) * p, 0)),
            pl.BlockSpec((_BO, 1), lambda p, i: (i * p, 0)),
            pl.BlockSpec((_C, _T), lambda p, i: (0, 0)),
            pl.BlockSpec((1, _C), lambda p, i: (0, 0)),
            pl.BlockSpec((1, 1), lambda p, i: (0, 0)),
        ],
        out_specs=pl.BlockSpec((_C, _K), lambda p, i: (0, 0)),
        out_shape=jax.ShapeDtypeStruct((_C, _K), jnp.float32),
        scratch_shapes=[
            pltpu.VMEM((_C, _T), jnp.float32),
            pltpu.VMEM((_C, _T), jnp.float32),
        ],
    )(w1t, w2t, b1.reshape(_C, 1), b2.reshape(_C, 1), X,
      Ws.reshape(1, _C), bs.reshape(1, 1))
    return out[None]


# speedup vs baseline: 5.5586x; 1.0297x over previous
import jax
import jax.numpy as jnp
from jax import lax
from jax.experimental import pallas as pl
from jax.experimental.pallas import tpu as pltpu

_T = 320
_C = 1024
_K = 48
_KPAD = 128
_BO = 512
_G = _C // _BO


def _conv3(w_ref, xv, b_ref):
    z = jnp.zeros((_C, 1), jnp.float32)
    xr = jnp.concatenate([z, xv[:, :_T - 1]], axis=1)
    xl = jnp.concatenate([xv[:, 1:], z], axis=1)
    acc = jnp.dot(w_ref[0], xr, preferred_element_type=jnp.float32)
    acc += jnp.dot(w_ref[1], xv, preferred_element_type=jnp.float32)
    acc += jnp.dot(w_ref[2], xl, preferred_element_type=jnp.float32)
    return jnp.maximum(acc + b_ref[...], 0.0)


def _tail_compute(h, ws, bs, o_ref):
    s = jnp.dot(ws.astype(jnp.bfloat16), h.astype(jnp.bfloat16),
                preferred_element_type=jnp.float32) + bs
    csum = jnp.sum(h, axis=0, keepdims=True)

    u_iota = lax.broadcasted_iota(jnp.int32, (_T, _T), 0)
    t_iota = lax.broadcasted_iota(jnp.int32, (_T, _T), 1)
    eye = (u_iota == t_iota).astype(jnp.float32)
    diag = eye * s
    scol = jnp.dot(diag, jnp.ones((_T, _T), jnp.float32),
                   preferred_element_type=jnp.float32,
                   precision=lax.Precision.HIGHEST)
    cmp = (scol >= s).astype(jnp.float32)
    cnt = jnp.sum(cmp, axis=0, keepdims=True)
    low = jnp.max(jnp.where(cnt >= _K, s, -1e30), axis=1, keepdims=True)

    maskf = (s >= low).astype(jnp.float32)
    nz = maskf * (csum != 0.0).astype(jnp.float32)
    ltm = (u_iota <= t_iota).astype(jnp.float32)
    prefix = jnp.dot(nz, ltm, preferred_element_type=jnp.float32,
                     precision=lax.Precision.HIGHEST)
    count = jnp.sum(nz, axis=1, keepdims=True)

    j_iota = lax.broadcasted_iota(jnp.int32, (_KPAD, _T), 0).astype(jnp.float32)
    t2 = lax.broadcasted_iota(jnp.int32, (_KPAD, _T), 1)
    sel = ((prefix == j_iota + 1.0) & (nz > 0.0)).astype(jnp.float32)
    pad = ((j_iota + 1.0 > count) & (t2 == 0)).astype(jnp.float32) * maskf
    pt = sel + pad
    g = lax.dot_general(h, pt, (((1,), (1,)), ((), ())),
                        preferred_element_type=jnp.float32,
                        precision=lax.Precision.HIGHEST)
    o_ref[...] = g[:, :_K]


def _fused_body(w1_ref, w2_ref, b1_ref, b2_ref, x_ref, ws_ref, bs_ref,
                o_ref, h1s, h2s):
    p = pl.program_id(0)
    i = pl.program_id(1)

    @pl.when(p == 0)
    def _():
        h1s[pl.ds(i * _BO, _BO), :] = _conv3(w1_ref, x_ref[...], b1_ref)

    @pl.when(p == 1)
    def _():
        h2s[pl.ds(i * _BO, _BO), :] = _conv3(w2_ref, h1s[...], b2_ref)

    @pl.when((p == 1) & (i == _G - 1))
    def _():
        _tail_compute(h2s[...], ws_ref[...], bs_ref[...], o_ref)


def kernel(x, W1, b1, W2, b2, Ws, bs):
    X = x[0]
    w1t = W1.transpose(2, 0, 1)
    w2t = W2.transpose(2, 0, 1)
    out = pl.pallas_call(
        _fused_body,
        grid=(2, _G),
        in_specs=[
            pl.BlockSpec((3, _BO, _C), lambda p, i: (0, i * (1 - p) + (_G - 1) * p, 0)),
            pl.BlockSpec((3, _BO, _C), lambda p, i: (0, i * p, 0)),
            pl.BlockSpec((_BO, 1), lambda p, i: (i * (1 - p) + (_G - 1) * p, 0)),
            pl.BlockSpec((_BO, 1), lambda p, i: (i * p, 0)),
            pl.BlockSpec((_C, _T), lambda p, i: (0, 0)),
            pl.BlockSpec((1, _C), lambda p, i: (0, 0)),
            pl.BlockSpec((1, 1), lambda p, i: (0, 0)),
        ],
        out_specs=pl.BlockSpec((_C, _K), lambda p, i: (0, 0)),
        out_shape=jax.ShapeDtypeStruct((_C, _K), jnp.float32),
        scratch_shapes=[
            pltpu.VMEM((_C, _T), jnp.float32),
            pltpu.VMEM((_C, _T), jnp.float32),
        ],
    )(w1t, w2t, b1.reshape(_C, 1), b2.reshape(_C, 1), X,
      Ws.reshape(1, _C), bs.reshape(1, 1))
    return out[None]


# speedup vs baseline: 5.5617x; 1.0006x over previous
import jax
import jax.numpy as jnp
from jax import lax
from jax.experimental import pallas as pl
from jax.experimental.pallas import tpu as pltpu

_T = 320
_C = 1024
_K = 48
_KPAD = 128
_BO = 512
_G = _C // _BO


def _conv3(wa_ref, wb_ref, xv, b_ref):
    z = jnp.zeros((_C, 1), jnp.float32)
    xr = jnp.concatenate([z, xv[:, :_T - 1]], axis=1)
    xl = jnp.concatenate([xv[:, 1:], z], axis=1)
    h = _C // 2
    acc = jnp.dot(wa_ref[0], xr[:h], preferred_element_type=jnp.float32)
    acc += jnp.dot(wa_ref[1], xv[:h], preferred_element_type=jnp.float32)
    acc += jnp.dot(wa_ref[2], xl[:h], preferred_element_type=jnp.float32)
    acc += jnp.dot(wb_ref[0], xr[h:], preferred_element_type=jnp.float32)
    acc += jnp.dot(wb_ref[1], xv[h:], preferred_element_type=jnp.float32)
    acc += jnp.dot(wb_ref[2], xl[h:], preferred_element_type=jnp.float32)
    return jnp.maximum(acc + b_ref[...], 0.0)


def _tail_compute(h, ws, bs, o_ref):
    s = jnp.dot(ws.astype(jnp.bfloat16), h.astype(jnp.bfloat16),
                preferred_element_type=jnp.float32) + bs
    csum = jnp.sum(h, axis=0, keepdims=True)

    u_iota = lax.broadcasted_iota(jnp.int32, (_T, _T), 0)
    t_iota = lax.broadcasted_iota(jnp.int32, (_T, _T), 1)
    eye = (u_iota == t_iota).astype(jnp.float32)
    diag = eye * s
    scol = jnp.dot(diag, jnp.ones((_T, _T), jnp.float32),
                   preferred_element_type=jnp.float32,
                   precision=lax.Precision.HIGHEST)
    cmp = (scol >= s).astype(jnp.float32)
    cnt = jnp.sum(cmp, axis=0, keepdims=True)
    low = jnp.max(jnp.where(cnt >= _K, s, -1e30), axis=1, keepdims=True)

    maskf = (s >= low).astype(jnp.float32)
    nz = maskf * (csum != 0.0).astype(jnp.float32)
    ltm = (u_iota <= t_iota).astype(jnp.float32)
    prefix = jnp.dot(nz, ltm, preferred_element_type=jnp.float32,
                     precision=lax.Precision.HIGHEST)
    count = jnp.sum(nz, axis=1, keepdims=True)

    j_iota = lax.broadcasted_iota(jnp.int32, (_KPAD, _T), 0).astype(jnp.float32)
    t2 = lax.broadcasted_iota(jnp.int32, (_KPAD, _T), 1)
    sel = ((prefix == j_iota + 1.0) & (nz > 0.0)).astype(jnp.float32)
    pad = ((j_iota + 1.0 > count) & (t2 == 0)).astype(jnp.float32) * maskf
    pt = sel + pad
    g = lax.dot_general(h, pt, (((1,), (1,)), ((), ())),
                        preferred_element_type=jnp.float32,
                        precision=lax.Precision.HIGHEST)
    o_ref[...] = g[:, :_K]


def _fused_body(w1a_ref, w1b_ref, w2a_ref, w2b_ref, b1_ref, b2_ref, x_ref,
                ws_ref, bs_ref, o_ref, h1s, h2s):
    p = pl.program_id(0)
    i = pl.program_id(1)

    @pl.when(p == 0)
    def _():
        h1s[pl.ds(i * _BO, _BO), :] = _conv3(w1a_ref, w1b_ref, x_ref[...], b1_ref)

    @pl.when(p == 1)
    def _():
        h2s[pl.ds(i * _BO, _BO), :] = _conv3(w2a_ref, w2b_ref, h1s[...], b2_ref)

    @pl.when((p == 1) & (i == _G - 1))
    def _():
        _tail_compute(h2s[...], ws_ref[...], bs_ref[...], o_ref)


def kernel(x, W1, b1, W2, b2, Ws, bs):
    X = x[0]
    w1t = W1.transpose(2, 0, 1)
    w2t = W2.transpose(2, 0, 1)
    out = pl.pallas_call(
        _fused_body,
        grid=(2, _G),
        in_specs=[
            pl.BlockSpec((3, _BO, _C // 2), lambda p, i: (0, i * (1 - p) + (_G - 1) * p, 0)),
            pl.BlockSpec((3, _BO, _C // 2), lambda p, i: (0, i * (1 - p) + (_G - 1) * p, 1)),
            pl.BlockSpec((3, _BO, _C // 2), lambda p, i: (0, i * p, 0)),
            pl.BlockSpec((3, _BO, _C // 2), lambda p, i: (0, i * p, 1)),
            pl.BlockSpec((_BO, 1), lambda p, i: (i * (1 - p) + (_G - 1) * p, 0)),
            pl.BlockSpec((_BO, 1), lambda p, i: (i * p, 0)),
            pl.BlockSpec((_C, _T), lambda p, i: (0, 0)),
            pl.BlockSpec((1, _C), lambda p, i: (0, 0)),
            pl.BlockSpec((1, 1), lambda p, i: (0, 0)),
        ],
        out_specs=pl.BlockSpec((_C, _K), lambda p, i: (0, 0)),
        out_shape=jax.ShapeDtypeStruct((_C, _K), jnp.float32),
        scratch_shapes=[
            pltpu.VMEM((_C, _T), jnp.float32),
            pltpu.VMEM((_C, _T), jnp.float32),
        ],
    )(w1t, w1t, w2t, w2t, b1.reshape(_C, 1), b2.reshape(_C, 1), X,
      Ws.reshape(1, _C), bs.reshape(1, 1))
    return out[None]


# speedup vs baseline: 5.5897x; 1.0050x over previous
import jax
import jax.numpy as jnp
from jax import lax
from jax.experimental import pallas as pl
from jax.experimental.pallas import tpu as pltpu

_T = 320
_C = 1024
_K = 48
_KPAD = 128
_BO = 512
_G = _C // _BO


def _conv3(w_ref, xv, b_ref):
    z = jnp.zeros((_C, 1), jnp.float32)
    xr = jnp.concatenate([z, xv[:, :_T - 1]], axis=1)
    xl = jnp.concatenate([xv[:, 1:], z], axis=1)
    acc = jnp.dot(w_ref[0], xr, preferred_element_type=jnp.float32)
    acc += jnp.dot(w_ref[1], xv, preferred_element_type=jnp.float32)
    acc += jnp.dot(w_ref[2], xl, preferred_element_type=jnp.float32)
    return jnp.maximum(acc + b_ref[...], 0.0)


def _tail_compute(h, ws, bs, o_ref):
    s = jnp.dot(ws.astype(jnp.bfloat16), h.astype(jnp.bfloat16),
                preferred_element_type=jnp.float32) + bs
    csum = jnp.sum(h, axis=0, keepdims=True)

    u_iota = lax.broadcasted_iota(jnp.int32, (_T, _T), 0)
    t_iota = lax.broadcasted_iota(jnp.int32, (_T, _T), 1)
    eye = (u_iota == t_iota).astype(jnp.float32)
    diag = eye * s
    scol = jnp.dot(diag, jnp.ones((_T, _T), jnp.float32),
                   preferred_element_type=jnp.float32,
                   precision=lax.Precision.HIGHEST)
    cmp = (scol >= s).astype(jnp.float32)
    cnt = jnp.sum(cmp, axis=0, keepdims=True)
    low = jnp.max(jnp.where(cnt >= _K, s, -1e30), axis=1, keepdims=True)

    maskf = (s >= low).astype(jnp.float32)
    nz = maskf * (csum != 0.0).astype(jnp.float32)
    ltm = (u_iota <= t_iota).astype(jnp.float32)
    prefix = jnp.dot(nz, ltm, preferred_element_type=jnp.float32,
                     precision=lax.Precision.HIGHEST)
    count = jnp.sum(nz, axis=1, keepdims=True)

    j_iota = lax.broadcasted_iota(jnp.int32, (_KPAD, _T), 0).astype(jnp.float32)
    t2 = lax.broadcasted_iota(jnp.int32, (_KPAD, _T), 1)
    sel = ((prefix == j_iota + 1.0) & (nz > 0.0)).astype(jnp.float32)
    pad = ((j_iota + 1.0 > count) & (t2 == 0)).astype(jnp.float32) * maskf
    pt = sel + pad
    g = lax.dot_general(h, pt, (((1,), (1,)), ((), ())),
                        preferred_element_type=jnp.float32,
                        precision=lax.Precision.HIGHEST)
    o_ref[...] = g[:, :_K]


def _fused_body(w1_ref, w2_ref, b1_ref, b2_ref, x_ref,
                ws_ref, bs_ref, o_ref, h1s, h2s):
    p = pl.program_id(0)
    i = pl.program_id(1)

    @pl.when(p == 0)
    def _():
        h1s[pl.ds(i * _BO, _BO), :] = _conv3(w1_ref, x_ref[...], b1_ref)

    @pl.when(p == 1)
    def _():
        h2s[pl.ds(i * _BO, _BO), :] = _conv3(w2_ref, h1s[...], b2_ref)

    @pl.when((p == 1) & (i == _G - 1))
    def _():
        _tail_compute(h2s[...], ws_ref[...], bs_ref[...], o_ref)


def kernel(x, W1, b1, W2, b2, Ws, bs):
    X = x[0]
    w1t = W1.transpose(2, 0, 1)
    w2t = W2.transpose(2, 0, 1)
    out = pl.pallas_call(
        _fused_body,
        grid=(2, _G),
        in_specs=[
            pl.BlockSpec((3, _BO, _C), lambda p, i: (0, i * (1 - p) + (_G - 1) * p, 0)),
            pl.BlockSpec((3, _BO, _C), lambda p, i: (0, i * p, 0)),
            pl.BlockSpec((_BO, 1), lambda p, i: (i * (1 - p) + (_G - 1) * p, 0)),
            pl.BlockSpec((_BO, 1), lambda p, i: (i * p, 0)),
            pl.BlockSpec((_C, _T), lambda p, i: (0, 0)),
            pl.BlockSpec((1, _C), lambda p, i: (0, 0)),
            pl.BlockSpec((1, 1), lambda p, i: (0, 0)),
        ],
        out_specs=pl.BlockSpec((_C, _K), lambda p, i: (0, 0)),
        out_shape=jax.ShapeDtypeStruct((_C, _K), jnp.float32),
        scratch_shapes=[
            pltpu.VMEM((_C, _T), jnp.float32),
            pltpu.VMEM((_C, _T), jnp.float32),
        ],
    )(w1t, w2t, b1.reshape(_C, 1), b2.reshape(_C, 1), X,
      Ws.reshape(1, _C), bs.reshape(1, 1))
    return out[None]
